# baseline (speedup 1.0000x reference)
import jax
import jax.numpy as jnp
from jax.experimental import pallas as pl
from jax.experimental.pallas import tpu as pltpu

BETA = 0.2
B, C, H, W = 8, 64, 64, 64
K, D = 1024, 64
N = B * H * W
BT = H * W


def _vq_block(feat_ref, cbh_ref, y2_ref, out_ref, part_ref):
    x = feat_ref[0].reshape(C, BT)
    cb_hi = cbh_ref[...]
    y2 = y2_ref[...]
    xs = (-2.0 * x).astype(jnp.bfloat16)
    scores = y2 + jax.lax.dot_general(
        cb_hi, xs, (((1,), (0,)), ((), ())),
        preferred_element_type=jnp.float32)
    smin = jnp.min(scores, axis=0)
    onehot = (scores == smin[None, :]).astype(jnp.bfloat16)
    res = jax.lax.dot_general(
        onehot, cb_hi, (((0,), (0,)), ((), ())),
        preferred_element_type=jnp.float32)
    out_ref[...] = res.reshape(1, C, H, W)
    part_ref[...] = jnp.full((1, 1, 1), jnp.sum(smin) + jnp.sum(x * x),
                             dtype=jnp.float32)


@jax.jit
def kernel(features, codebook):
    y2 = jnp.sum(codebook * codebook, axis=1, keepdims=True)
    cb_hi = codebook.astype(jnp.bfloat16)
    out, parts = pl.pallas_call(
        _vq_block,
        grid=(B,),
        in_specs=[
            pl.BlockSpec((1, C, H, W), lambda b: (b, 0, 0, 0)),
            pl.BlockSpec((K, D), lambda b: (0, 0)),
            pl.BlockSpec((K, 1), lambda b: (0, 0)),
        ],
        out_specs=[
            pl.BlockSpec((1, C, H, W), lambda b: (b, 0, 0, 0)),
            pl.BlockSpec((1, 1, 1), lambda b: (b, 0, 0)),
        ],
        out_shape=[
            jax.ShapeDtypeStruct((B, C, H, W), jnp.float32),
            jax.ShapeDtypeStruct((B, 1, 1), jnp.float32),
        ],
    )(features, cb_hi, y2)
    loss = jnp.sum(parts) * ((1.0 + BETA) / (N * D))
    return out, loss


# speedup vs baseline: 1.0718x; 1.0718x over previous
import jax
import jax.numpy as jnp
from jax.experimental import pallas as pl
from jax.experimental.pallas import tpu as pltpu

BETA = 0.2
B, C, H, W = 8, 64, 64, 64
K, D = 1024, 64
N = B * H * W
BT = H * W


def _vq_block(feat_ref, cb_ref, out_ref, part_ref):
    x = feat_ref[0].reshape(C, BT)
    cb = cb_ref[...]
    cb_hi = cb.astype(jnp.bfloat16)
    y2 = jnp.sum(cb * cb, axis=1, keepdims=True)
    xs = (-2.0 * x).astype(jnp.bfloat16)
    scores = y2 + jax.lax.dot_general(
        cb_hi, xs, (((1,), (0,)), ((), ())),
        preferred_element_type=jnp.float32)
    smin = jnp.min(scores, axis=0)
    onehot = (scores == smin[None, :]).astype(jnp.bfloat16)
    res = jax.lax.dot_general(
        onehot, cb_hi, (((0,), (0,)), ((), ())),
        preferred_element_type=jnp.float32)
    out_ref[...] = res.reshape(1, C, H, W)
    part_ref[...] = jnp.full((1, 1, 1), jnp.sum(smin) + jnp.sum(x * x),
                             dtype=jnp.float32)


@jax.jit
def kernel(features, codebook):
    out, parts = pl.pallas_call(
        _vq_block,
        grid=(B,),
        in_specs=[
            pl.BlockSpec((1, C, H, W), lambda b: (b, 0, 0, 0)),
            pl.BlockSpec((K, D), lambda b: (0, 0)),
        ],
        out_specs=[
            pl.BlockSpec((1, C, H, W), lambda b: (b, 0, 0, 0)),
            pl.BlockSpec((1, 1, 1), lambda b: (b, 0, 0)),
        ],
        out_shape=[
            jax.ShapeDtypeStruct((B, C, H, W), jnp.float32),
            jax.ShapeDtypeStruct((B, 1, 1), jnp.float32),
        ],
    )(features, codebook)
    loss = jnp.sum(parts) * ((1.0 + BETA) / (N * D))
    return out, loss
